# Initial kernel scaffold; baseline (speedup 1.0000x reference)
import jax
import jax.numpy as jnp
from jax.experimental import pallas as pl


def kernel(xyz):
    raise NotImplementedError("write your pallas kernel here")


# speedup vs baseline: 1.9032x; 1.9032x over previous
import functools

import jax
import jax.numpy as jnp
from jax import lax
from jax.experimental import pallas as pl
from jax.experimental.pallas import tpu as pltpu

B = 8
N = 8192
G = 512
K = 32


def _fps_body(xs_ref, ys_ref, zs_ref, idx_ref, cx_ref, cy_ref, cz_ref, dists_ref):
    xs = xs_ref[...]
    ys = ys_ref[...]
    zs = zs_ref[...]
    lanes = lax.broadcasted_iota(jnp.int32, (B, N), 1)
    glanes = lax.broadcasted_iota(jnp.int32, (B, G), 1)
    dists_ref[...] = jnp.full((B, N), jnp.inf, dtype=jnp.float32)

    idx_ref[...] = jnp.zeros((B, G), jnp.int32)
    cx_ref[...] = jnp.zeros((B, G), jnp.float32)
    cy_ref[...] = jnp.zeros((B, G), jnp.float32)
    cz_ref[...] = jnp.zeros((B, G), jnp.float32)

    def step(i, farthest):
        onehot = lanes == farthest
        cx = jnp.sum(jnp.where(onehot, xs, 0.0), axis=1, keepdims=True)
        cy = jnp.sum(jnp.where(onehot, ys, 0.0), axis=1, keepdims=True)
        cz = jnp.sum(jnp.where(onehot, zs, 0.0), axis=1, keepdims=True)
        slot = glanes == i
        slot_i = slot.astype(jnp.int32)
        slot_f = slot.astype(jnp.float32)
        idx_ref[...] = idx_ref[...] + slot_i * farthest
        cx_ref[...] = cx_ref[...] + slot_f * cx
        cy_ref[...] = cy_ref[...] + slot_f * cy
        cz_ref[...] = cz_ref[...] + slot_f * cz
        dx = xs - cx
        dy = ys - cy
        dz = zs - cz
        d = (dx * dx + dy * dy) + dz * dz
        dists = jnp.minimum(dists_ref[...], d)
        dists_ref[...] = dists
        m = jnp.max(dists, axis=1, keepdims=True)
        far_new = jnp.min(jnp.where(dists == m, lanes, N), axis=1, keepdims=True)
        return far_new

    lax.fori_loop(0, G, step, jnp.zeros((B, 1), jnp.int32))


def _fps_call(xs, ys, zs, interpret=False):
    return pl.pallas_call(
        _fps_body,
        out_shape=(
            jax.ShapeDtypeStruct((B, G), jnp.int32),
            jax.ShapeDtypeStruct((B, G), jnp.float32),
            jax.ShapeDtypeStruct((B, G), jnp.float32),
            jax.ShapeDtypeStruct((B, G), jnp.float32),
        ),
        scratch_shapes=[pltpu.VMEM((B, N), jnp.float32)],
        interpret=interpret,
    )(xs, ys, zs)


def _knn_jax(xs, ys, zs, cx, cy, cz):
    center = jnp.stack([cx, cy, cz], axis=-1)
    xyz_only = jnp.stack([xs, ys, zs], axis=-1)
    q2 = jnp.sum(center ** 2, axis=-1)[:, :, None]
    r2 = jnp.sum(xyz_only ** 2, axis=-1)[:, None, :]
    d2 = q2 + r2 - 2.0 * jnp.einsum('bgc,bnc->bgn', center, xyz_only)
    _, idx = lax.top_k(-d2, K)
    idx_base = (jnp.arange(B, dtype=idx.dtype)[:, None, None]) * N
    idx_flat = (idx + idx_base).reshape(-1)
    neigh_xyz = xyz_only.reshape(B * N, 3)[idx_flat, :].reshape(B, G, K, 3)
    neighborhood = neigh_xyz - center[:, :, None, :]
    return neighborhood, center, idx


def kernel(xyz):
    xs = xyz[:, :, 0]
    ys = xyz[:, :, 1]
    zs = xyz[:, :, 2]
    fps_idx, cx, cy, cz = _fps_call(xs, ys, zs)
    return _knn_jax(xs, ys, zs, cx, cy, cz)


# speedup vs baseline: 16.6403x; 8.7436x over previous
import functools

import jax
import jax.numpy as jnp
from jax import lax
from jax.experimental import pallas as pl
from jax.experimental.pallas import tpu as pltpu
from jax.experimental.pallas import tpu_sc as plsc

B = 8
N = 8192
G = 512
K = 32

NC = 2
NS = 16
NW = NC * NS
QPW = (B * G) // NW
WPB = G // QPW
NCHUNK = N // 16
NGROUP = NCHUNK // 16


def _fps_body(xs_ref, ys_ref, zs_ref, idx_ref, cx_ref, cy_ref, cz_ref, dists_ref):
    xs = xs_ref[...]
    ys = ys_ref[...]
    zs = zs_ref[...]
    lanes = lax.broadcasted_iota(jnp.int32, (B, N), 1)
    glanes = lax.broadcasted_iota(jnp.int32, (B, G), 1)
    dists_ref[...] = jnp.full((B, N), jnp.inf, dtype=jnp.float32)
    idx_ref[...] = jnp.zeros((B, G), jnp.int32)
    cx_ref[...] = jnp.zeros((B, G), jnp.float32)
    cy_ref[...] = jnp.zeros((B, G), jnp.float32)
    cz_ref[...] = jnp.zeros((B, G), jnp.float32)

    def step(i, farthest):
        onehot = lanes == farthest
        cx = jnp.sum(jnp.where(onehot, xs, 0.0), axis=1, keepdims=True)
        cy = jnp.sum(jnp.where(onehot, ys, 0.0), axis=1, keepdims=True)
        cz = jnp.sum(jnp.where(onehot, zs, 0.0), axis=1, keepdims=True)
        slot = glanes == i
        slot_i = slot.astype(jnp.int32)
        slot_f = slot.astype(jnp.float32)
        idx_ref[...] = idx_ref[...] + slot_i * farthest
        cx_ref[...] = cx_ref[...] + slot_f * cx
        cy_ref[...] = cy_ref[...] + slot_f * cy
        cz_ref[...] = cz_ref[...] + slot_f * cz
        dx = xs - cx
        dy = ys - cy
        dz = zs - cz
        d = (dx * dx + dy * dy) + dz * dz
        dists = jnp.minimum(dists_ref[...], d)
        dists_ref[...] = dists
        m = jnp.max(dists, axis=1, keepdims=True)
        far_new = jnp.min(jnp.where(dists == m, lanes, N), axis=1, keepdims=True)
        return far_new

    lax.fori_loop(0, G, step, jnp.zeros((B, 1), jnp.int32))


def _fps_call(xs, ys, zs, interpret=False):
    return pl.pallas_call(
        _fps_body,
        out_shape=(
            jax.ShapeDtypeStruct((B, G), jnp.int32),
            jax.ShapeDtypeStruct((B, G), jnp.float32),
            jax.ShapeDtypeStruct((B, G), jnp.float32),
            jax.ShapeDtypeStruct((B, G), jnp.float32),
        ),
        scratch_shapes=[pltpu.VMEM((B, N), jnp.float32)],
        interpret=interpret,
    )(xs, ys, zs)


def _knn_sc_body(xs_hbm, ys_hbm, zs_hbm, cx_hbm, cy_hbm, cz_hbm,
                 idx_out, ngx_out, ngy_out, ngz_out,
                 xs_t, ys_t, zs_t, xr_t, yr_t, zr_t, r2_t, d_t,
                 cx_t, cy_t, cz_t, q2_t,
                 colmin_t, idx_st, ngx_st, ngy_st, ngz_st):
    wid = lax.axis_index("s") * NC + lax.axis_index("c")
    b = wid // WPB
    qb = (wid % WPB) * QPW
    pltpu.sync_copy(xs_hbm.at[b], xs_t)
    pltpu.sync_copy(ys_hbm.at[b], ys_t)
    pltpu.sync_copy(zs_hbm.at[b], zs_t)
    pltpu.sync_copy(cx_hbm.at[b, pl.ds(qb, QPW)], cx_t)
    pltpu.sync_copy(cy_hbm.at[b, pl.ds(qb, QPW)], cy_t)
    pltpu.sync_copy(cz_hbm.at[b, pl.ds(qb, QPW)], cz_t)

    iota = lax.broadcasted_iota(jnp.int32, (16,), 0)
    INF = jnp.float32(jnp.inf)
    BIG = jnp.int32(1 << 30)

    _gdn = lax.GatherDimensionNumbers(
        offset_dims=(), collapsed_slice_dims=(0,), start_index_map=(0,))

    def _shuf(v, idx):
        return lax.gather(v, idx[:, None], dimension_numbers=_gdn,
                          slice_sizes=(1,),
                          mode=lax.GatherScatterMode.PROMISE_IN_BOUNDS)

    def _bmin(v):
        for s in (1, 2, 4, 8):
            v = jnp.minimum(v, _shuf(v, iota ^ s))
        return v

    def _pick(ref, qi):
        v = ref[pl.ds((qi // 16) * 16, 16)]
        return _shuf(v, iota * 0 + (qi % 16))

    def _bf16r(v):
        C = jnp.float32(65537.0)
        x = C * v
        y = x - v
        return x - y

    def r2_chunk(j, _):
        for u in range(4):
            off = j * 64 + u * 16
            px = xs_t[pl.ds(off, 16)]
            py = ys_t[pl.ds(off, 16)]
            pz = zs_t[pl.ds(off, 16)]
            r2_t[pl.ds(off, 16)] = (px * px + py * py) + pz * pz
            xr_t[pl.ds(off, 16)] = _bf16r(px)
            yr_t[pl.ds(off, 16)] = _bf16r(py)
            zr_t[pl.ds(off, 16)] = _bf16r(pz)
        return 0
    lax.fori_loop(0, NCHUNK // 4, r2_chunk, 0)

    def q2_chunk(j, _):
        cxv = cx_t[pl.ds(j * 16, 16)]
        cyv = cy_t[pl.ds(j * 16, 16)]
        czv = cz_t[pl.ds(j * 16, 16)]
        q2_t[pl.ds(j * 16, 16)] = (cxv * cxv + cyv * cyv) + czv * czv
        return 0
    lax.fori_loop(0, QPW // 16, q2_chunk, 0)

    def per_query(qi, _):
        cxq = _pick(cx_t, qi)
        cyq = _pick(cy_t, qi)
        czq = _pick(cz_t, qi)
        q2 = _pick(q2_t, qi)
        cxr = _bf16r(cxq)
        cyr = _bf16r(cyq)
        czr = _bf16r(czq)

        def dist_group(g, _):
            rmin = iota.astype(jnp.float32) * 0 + INF
            for u in range(16):
                off = g * 256 + u * 16
                px = xr_t[pl.ds(off, 16)]
                py = yr_t[pl.ds(off, 16)]
                pz = zr_t[pl.ds(off, 16)]
                dot = (cxr * px + cyr * py) + czr * pz
                d = (q2 + r2_t[pl.ds(off, 16)]) - 2.0 * dot
                d_t[pl.ds(off, 16)] = d
                rmin = jnp.minimum(rmin, d)
            colmin_t[pl.ds(g * 16, 16)] = rmin
            return 0
        lax.fori_loop(0, NGROUP, dist_group, 0)

        def sel(k, carry):
            na0, na1, nx0, nx1, ny0, ny1, nz0, nz1 = carry

            def scan(jj, c2):
                acc, idxa = c2
                for u in range(2):
                    p = jj * 2 + u
                    v = colmin_t[pl.ds(p * 16, 16)]
                    m = v < acc
                    acc = jnp.where(m, v, acc)
                    idxa = jnp.where(m, iota * 0 + p, idxa)
                return (acc, idxa)
            acc, idxa = lax.fori_loop(0, NGROUP // 2, scan,
                                      (iota.astype(jnp.float32) * 0 + INF, iota * 0))
            gm = _bmin(acc)
            pos = _bmin(jnp.where(acc == gm, idxa * 16 + iota, BIG))
            gg_s = pos[0] >> 4

            nacc = iota * 0 + BIG
            rmin = iota.astype(jnp.float32) * 0 + INF
            for c16 in range(16):
                off = gg_s * 256 + c16 * 16
                dc = d_t[pl.ds(off, 16)]
                hit = dc == gm
                gidx = (gg_s * 16 + c16) * 16 + iota
                nacc = jnp.minimum(nacc, jnp.where(hit, gidx, BIG))
                upd = jnp.where(hit, INF, dc)
                d_t[pl.ds(off, 16)] = upd
                rmin = jnp.minimum(rmin, upd)
            colmin_t[pl.ds(gg_s * 16, 16)] = rmin
            n = _bmin(nacc)
            n_s = n[0]

            lane = n & 15
            coff = (n_s >> 4) * 16
            gx = _shuf(xs_t[pl.ds(coff, 16)], lane) - cxq
            gy = _shuf(ys_t[pl.ds(coff, 16)], lane) - cyq
            gz = _shuf(zs_t[pl.ds(coff, 16)], lane) - czq
            in0 = iota == k
            in1 = iota == (k - 16)
            na0 = jnp.where(in0, n, na0)
            na1 = jnp.where(in1, n, na1)
            nx0 = jnp.where(in0, gx, nx0)
            nx1 = jnp.where(in1, gx, nx1)
            ny0 = jnp.where(in0, gy, ny0)
            ny1 = jnp.where(in1, gy, ny1)
            nz0 = jnp.where(in0, gz, nz0)
            nz1 = jnp.where(in1, gz, nz1)
            return (na0, na1, nx0, nx1, ny0, ny1, nz0, nz1)

        zf = iota.astype(jnp.float32) * 0
        na0, na1, nx0, nx1, ny0, ny1, nz0, nz1 = lax.fori_loop(
            0, K, sel, (iota * 0, iota * 0, zf, zf, zf, zf, zf, zf))

        qoff = qi * K
        idx_st[pl.ds(qoff, 16)] = na0
        idx_st[pl.ds(qoff + 16, 16)] = na1
        ngx_st[pl.ds(qoff, 16)] = nx0
        ngx_st[pl.ds(qoff + 16, 16)] = nx1
        ngy_st[pl.ds(qoff, 16)] = ny0
        ngy_st[pl.ds(qoff + 16, 16)] = ny1
        ngz_st[pl.ds(qoff, 16)] = nz0
        ngz_st[pl.ds(qoff + 16, 16)] = nz1
        return 0
    lax.fori_loop(0, QPW, per_query, 0)

    pltpu.sync_copy(idx_st, idx_out.at[b, pl.ds(qb * K, QPW * K)])
    pltpu.sync_copy(ngx_st, ngx_out.at[b, pl.ds(qb * K, QPW * K)])
    pltpu.sync_copy(ngy_st, ngy_out.at[b, pl.ds(qb * K, QPW * K)])
    pltpu.sync_copy(ngz_st, ngz_out.at[b, pl.ds(qb * K, QPW * K)])


def _knn_sc_call(xs, ys, zs, cx, cy, cz, interpret=False):
    mesh = plsc.VectorSubcoreMesh(core_axis_name="c", subcore_axis_name="s")
    f = pl.kernel(
        _knn_sc_body,
        out_type=(
            jax.ShapeDtypeStruct((B, G * K), jnp.int32),
            jax.ShapeDtypeStruct((B, G * K), jnp.float32),
            jax.ShapeDtypeStruct((B, G * K), jnp.float32),
            jax.ShapeDtypeStruct((B, G * K), jnp.float32),
        ),
        mesh=mesh,
        scratch_types=[
            pltpu.VMEM((N,), jnp.float32),
            pltpu.VMEM((N,), jnp.float32),
            pltpu.VMEM((N,), jnp.float32),
            pltpu.VMEM((N,), jnp.float32),
            pltpu.VMEM((N,), jnp.float32),
            pltpu.VMEM((N,), jnp.float32),
            pltpu.VMEM((N,), jnp.float32),
            pltpu.VMEM((N,), jnp.float32),
            pltpu.VMEM((QPW,), jnp.float32),
            pltpu.VMEM((QPW,), jnp.float32),
            pltpu.VMEM((QPW,), jnp.float32),
            pltpu.VMEM((QPW,), jnp.float32),
            pltpu.VMEM((NCHUNK,), jnp.float32),
            pltpu.VMEM((QPW * K,), jnp.int32),
            pltpu.VMEM((QPW * K,), jnp.float32),
            pltpu.VMEM((QPW * K,), jnp.float32),
            pltpu.VMEM((QPW * K,), jnp.float32),
        ],
        interpret=interpret,
    )
    idx_flat, ngx, ngy, ngz = f(xs, ys, zs, cx, cy, cz)
    idx = idx_flat.reshape(B, G, K)
    ngh = jnp.stack([ngx.reshape(B, G, K), ngy.reshape(B, G, K),
                     ngz.reshape(B, G, K)], axis=-1)
    return idx, ngh


def kernel(xyz):
    xs = xyz[:, :, 0]
    ys = xyz[:, :, 1]
    zs = xyz[:, :, 2]
    fps_idx, cx, cy, cz = _fps_call(xs, ys, zs)
    idx, neighborhood = _knn_sc_call(xs, ys, zs, cx, cy, cz)
    center = jnp.stack([cx, cy, cz], axis=-1)
    return neighborhood, center, idx
